# baseline (speedup 1.0000x reference)
import functools

import jax
import jax.numpy as jnp
from jax import lax
from jax.experimental import pallas as pl
from jax.experimental.pallas import tpu as pltpu
from jax.experimental.pallas import tpu_sc as plsc

N = 10000
NPAD = 10112
E = 320000
CHUNK = 128
C = 72
NSC = 2
NSUB = 16
NTILES = NSC * NSUB
NCH = 140
EPT = NCH * C
EPAD = EPT * NTILES
EALLOC = EPAD + 2 * C
STRIPE = NPAD // NSUB


_PIECES = [(k * CHUNK, min(CHUNK, STRIPE - k * CHUNK))
           for k in range((STRIPE + CHUNK - 1) // CHUNK)]


def _make_edge_scatter(d, nch0=NCH, nch1=NCH):
    mesh = plsc.VectorSubcoreMesh(core_axis_name="c", subcore_axis_name="s")
    out_type = [jax.ShapeDtypeStruct((NSC, NPAD, d), jnp.float32)]
    scratch = [
        pltpu.VMEM_SHARED((NPAD, d), jnp.float32),
        [pltpu.VMEM((C,), jnp.int32) for _ in range(4)],
        [pltpu.VMEM((C,), jnp.int32) for _ in range(4)],
        [pltpu.VMEM((C, d), jnp.float32) for _ in range(2)],
        pltpu.VMEM((CHUNK, d), jnp.float32),
        pltpu.SemaphoreType.DMA,
        [pltpu.SemaphoreType.DMA for _ in range(2)],
        [pltpu.SemaphoreType.DMA for _ in range(4)],
    ]

    def body(y, srcp, dstp, zrows, out, acc, sidx, didx, rows, zbuf,
             semg, sems, semi):
        c = lax.axis_index("c")
        s = lax.axis_index("s")
        wid = c * NSUB + s
        r0 = s * STRIPE
        pltpu.sync_copy(zrows.at[pl.ds(0, CHUNK)], zbuf)
        for off, ln in _PIECES:
            pltpu.sync_copy(zbuf.at[pl.ds(0, ln)], acc.at[pl.ds(r0 + off, ln)])
        plsc.subcore_barrier()

        base = jnp.where(c == 0, s * (C * nch0),
                         NSUB * C * nch0 + s * (C * nch1))
        nquad = jnp.where(c == 0, nch0 // 4, nch1 // 4)

        def issue_idx(p, chunk):
            b = base + chunk * C
            pltpu.async_copy(srcp.at[pl.ds(b, C)], sidx[p], semi[p])
            pltpu.async_copy(dstp.at[pl.ds(b, C)], didx[p], semi[p])

        def wait_idx(p):
            pltpu.make_async_copy(srcp.at[pl.ds(0, C)], sidx[p], semi[p]).wait()
            pltpu.make_async_copy(dstp.at[pl.ds(0, C)], didx[p], semi[p]).wait()

        def wait_scat(x):
            pltpu.make_async_copy(rows[x], acc.at[didx[0]], sems[x]).wait()

        for p in range(4):
            issue_idx(p, p)

        def quad(k, carry):
            for j in range(4):
                cix = 4 * k + j
                x = j % 2
                if j >= 2:
                    wait_scat(x)
                    issue_idx(j - 2, cix + 2)
                else:
                    @pl.when(k > 0)
                    def _():
                        wait_scat(x)
                        issue_idx(j + 2, cix + 2)
                wait_idx(j)
                pltpu.async_copy(y.at[sidx[j]], rows[x], semg).wait()
                pltpu.async_copy(rows[x], acc.at[didx[j]], sems[x], add=True)
            return carry

        lax.fori_loop(0, nquad, quad, 0)
        wait_scat(0)
        wait_scat(1)
        wait_idx(0)
        wait_idx(1)
        plsc.subcore_barrier()

        for off, ln in _PIECES:
            pltpu.sync_copy(acc.at[pl.ds(r0 + off, ln)], zbuf.at[pl.ds(0, ln)])
            pltpu.sync_copy(zbuf.at[pl.ds(0, ln)],
                            out.at[c, pl.ds(r0 + off, ln)])

    return pl.kernel(body, out_type=out_type, mesh=mesh, scratch_types=scratch)


CW = 128


def _make_cnt_kernel():
    mesh = plsc.VectorSubcoreMesh(core_axis_name="c", subcore_axis_name="s")
    out_type = [jax.ShapeDtypeStruct((NSC, NPAD, CW), jnp.float32)]
    scratch = [
        pltpu.VMEM_SHARED((NPAD, CW), jnp.float32),
        [pltpu.VMEM((C,), jnp.int32) for _ in range(4)],
        pltpu.VMEM((C, CW), jnp.float32),
        pltpu.VMEM((CHUNK, CW), jnp.float32),
        [pltpu.SemaphoreType.DMA for _ in range(4)],
        [pltpu.SemaphoreType.DMA for _ in range(2)],
    ]

    def body(dstp, z16, ones_in, cnt_out, acc16, didx, ones_v, b16,
             semi, sems):
        c = lax.axis_index("c")
        s = lax.axis_index("s")
        wid = c * NSUB + s
        r0 = s * STRIPE
        pltpu.sync_copy(z16.at[pl.ds(0, CHUNK)], b16)
        for off, ln in _PIECES:
            pltpu.sync_copy(b16.at[pl.ds(0, ln)], acc16.at[pl.ds(r0 + off, ln)])
        pltpu.sync_copy(ones_in, ones_v)
        plsc.subcore_barrier()

        base = wid * EPT

        def issue_idx(p, chunk):
            pltpu.async_copy(dstp.at[pl.ds(base + chunk * C, C)],
                             didx[p], semi[p])

        def wait_idx(p):
            pltpu.make_async_copy(dstp.at[pl.ds(0, C)], didx[p],
                                  semi[p]).wait()

        def wait_scat(x):
            pltpu.make_async_copy(ones_v, acc16.at[didx[0]], sems[x]).wait()

        for p in range(4):
            issue_idx(p, p)

        def quad(k, carry):
            for j in range(4):
                cix = 4 * k + j
                x = j % 2
                if j >= 2:
                    wait_scat(x)
                    issue_idx(j - 2, cix + 2)
                else:
                    @pl.when(k > 0)
                    def _():
                        wait_scat(x)
                        issue_idx(j + 2, cix + 2)
                wait_idx(j)
                pltpu.async_copy(ones_v, acc16.at[didx[j]], sems[x], add=True)
            return carry

        lax.fori_loop(0, NCH // 4, quad, 0)
        wait_scat(0)
        wait_scat(1)
        wait_idx(0)
        wait_idx(1)
        plsc.subcore_barrier()

        for off, ln in _PIECES:
            pltpu.sync_copy(acc16.at[pl.ds(r0 + off, ln)], b16.at[pl.ds(0, ln)])
            pltpu.sync_copy(b16.at[pl.ds(0, ln)],
                            cnt_out.at[c, pl.ds(r0 + off, ln)])

    return pl.kernel(body, out_type=out_type, mesh=mesh, scratch_types=scratch)


def _softplus(x):
    return jnp.maximum(x, 0.0) + jnp.log(1.0 + jnp.exp(-jnp.abs(x)))


def _mish(x):
    return x * jnp.tanh(_softplus(x))


def _tc_prep_body(x_ref, wl_ref, wr_ref, bl_ref, y_ref, z_ref):
    xx = x_ref[...]
    yv = jnp.dot(xx, wl_ref[...], preferred_element_type=jnp.float32)
    y_ref[0:N] = yv
    y_ref[N:2 * N] = yv
    z_ref[...] = (jnp.dot(xx, wr_ref[...], preferred_element_type=jnp.float32)
                  + bl_ref[...])


def _norm_act(s_ref, c_ref, z_ref, g_ref, b_ref):
    s = s_ref[...]
    cc = c_ref[...]
    cnt = cc[0, :N, :1] + cc[1, :N, :1]
    h = (s[0, :N] + s[1, :N]) / jnp.maximum(cnt, 1.0) + z_ref[...]
    m = jnp.mean(h, axis=0, keepdims=True)
    v = jnp.mean((h - m) ** 2, axis=0, keepdims=True)
    return (h - m) * lax.rsqrt(v + 1e-5) * g_ref[...] + b_ref[...]


def _tc_mid_body(s_ref, c_ref, z_ref, g_ref, b_ref, wl_ref, wr_ref, bl_ref,
                 y_ref, z2_ref):
    a = _mish(_norm_act(s_ref, c_ref, z_ref, g_ref, b_ref))
    yv = jnp.dot(a, wl_ref[...], preferred_element_type=jnp.float32)
    y_ref[0:N] = yv
    y_ref[N:2 * N] = yv
    z2_ref[...] = (jnp.dot(a, wr_ref[...], preferred_element_type=jnp.float32)
                   + bl_ref[...])


def _tc_fin_body(s_ref, c_ref, z_ref, g_ref, b_ref,
                 pw1_ref, pb1_ref, pw2_ref, pb2_ref,
                 aw1_ref, ab1_ref, aw2_ref, ab2_ref,
                 pred_ref, emb_ref, aux_ref):
    s = s_ref[...]
    cc = c_ref[...]
    cnt = cc[0, :N, :1] + cc[1, :N, :1]
    h = (s[0, :N, :64] + s[1, :N, :64]) / jnp.maximum(cnt, 1.0) + z_ref[...]
    m = jnp.mean(h, axis=0, keepdims=True)
    v = jnp.mean((h - m) ** 2, axis=0, keepdims=True)
    emb = (h - m) * lax.rsqrt(v + 1e-5) * g_ref[...] + b_ref[...]
    emb_ref[...] = emb
    t = _mish(jnp.dot(emb, pw1_ref[...], preferred_element_type=jnp.float32)
              + pb1_ref[...])
    pred_ref[...] = (jnp.dot(t, pw2_ref[...], preferred_element_type=jnp.float32)
                     + pb2_ref[...])
    u = _mish(jnp.dot(emb, aw1_ref[...], preferred_element_type=jnp.float32)
              + ab1_ref[...])
    aux_ref[...] = (jnp.dot(u, aw2_ref[...], preferred_element_type=jnp.float32)
                    + ab2_ref[...])


def _f32(*shape):
    return jax.ShapeDtypeStruct(shape, jnp.float32)


def kernel(x, edge_index, conv0_Wl, conv0_bl, conv0_Wr, bn0_g, bn0_b,
           conv1_Wl, conv1_bl, conv1_Wr, bn1_g, bn1_b,
           conv2_Wl, conv2_bl, conv2_Wr, bn2_g, bn2_b,
           pred_W1, pred_b1, pred_W2, pred_b2,
           aux_W1, aux_b1, aux_W2, aux_b2):
    src = edge_index[0].astype(jnp.int32)
    dst = edge_index[1].astype(jnp.int32)
    pad = EALLOC - E
    src_p = jnp.concatenate([src, jnp.zeros((pad,), jnp.int32)])
    dst_p = jnp.concatenate([dst, jnp.full((pad,), N, jnp.int32)])
    b0 = NSUB * C * 200
    src_p = src_p + jnp.where(jnp.arange(EALLOC) < b0, 0, N).astype(jnp.int32)
    z128 = jnp.zeros((NPAD, 128), jnp.float32)
    zcw = jnp.zeros((NPAD, CW), jnp.float32)
    onescw = jnp.ones((C, CW), jnp.float32)

    scat = _make_edge_scatter(128, nch0=200, nch1=80)
    cntk = _make_cnt_kernel()

    (c0,) = cntk(dst_p, zcw, onescw)

    y0, z0 = pl.pallas_call(
        _tc_prep_body,
        out_shape=[_f32(2 * N, 128), _f32(N, 128)],
    )(x, conv0_Wl.T, conv0_Wr.T, conv0_bl[None, :])

    (s0,) = scat(y0, src_p, dst_p, z128)

    y1, z1 = pl.pallas_call(
        _tc_mid_body,
        out_shape=[_f32(2 * N, 128), _f32(N, 128)],
    )(s0, c0, z0, bn0_g[None, :], bn0_b[None, :], conv1_Wl.T, conv1_Wr.T,
      conv1_bl[None, :])

    (s1,) = scat(y1, src_p, dst_p, z128)

    wl2T_pad = jnp.pad(conv2_Wl.T, ((0, 0), (0, 64)))
    y2, z2 = pl.pallas_call(
        _tc_mid_body,
        out_shape=[_f32(2 * N, 128), _f32(N, 64)],
    )(s1, c0, z1, bn1_g[None, :], bn1_b[None, :], wl2T_pad, conv2_Wr.T,
      conv2_bl[None, :])

    (s2,) = scat(y2, src_p, dst_p, z128)

    pred, emb, aux = pl.pallas_call(
        _tc_fin_body,
        out_shape=[_f32(N, 1), _f32(N, 64), _f32(N, 1)],
    )(s2, c0, z2, bn2_g[None, :], bn2_b[None, :],
      pred_W1.T, pred_b1[None, :], pred_W2.T, pred_b2[None, :],
      aux_W1.T, aux_b1[None, :], aux_W2.T, aux_b2[None, :])

    return (pred, emb, aux)


# speedup vs baseline: 1.0220x; 1.0220x over previous
import functools

import jax
import jax.numpy as jnp
from jax import lax
from jax.experimental import pallas as pl
from jax.experimental.pallas import tpu as pltpu
from jax.experimental.pallas import tpu_sc as plsc

N = 10000
NPAD = 10112
E = 320000
CHUNK = 128
C = 72
NSC = 2
NSUB = 16
NTILES = NSC * NSUB
NCH = 140
EPT = NCH * C
EPAD = EPT * NTILES
EALLOC = EPAD + 2 * C
STRIPE = NPAD // NSUB


_PIECES = [(k * CHUNK, min(CHUNK, STRIPE - k * CHUNK))
           for k in range((STRIPE + CHUNK - 1) // CHUNK)]


def _make_edge_scatter(d, nch0=NCH, nch1=NCH):
    mesh = plsc.VectorSubcoreMesh(core_axis_name="c", subcore_axis_name="s")
    out_type = [jax.ShapeDtypeStruct((NSC, NPAD, d), jnp.float32)]
    scratch = [
        pltpu.VMEM_SHARED((NPAD, d), jnp.float32),
        [pltpu.VMEM((C,), jnp.int32) for _ in range(4)],
        [pltpu.VMEM((C,), jnp.int32) for _ in range(4)],
        [pltpu.VMEM((C, d), jnp.float32) for _ in range(2)],
        pltpu.VMEM((CHUNK, d), jnp.float32),
        pltpu.SemaphoreType.DMA,
        [pltpu.SemaphoreType.DMA for _ in range(2)],
        [pltpu.SemaphoreType.DMA for _ in range(4)],
    ]

    def body(y, srcp, dstp, zrows, out, acc, sidx, didx, rows, zbuf,
             semg, sems, semi):
        c = lax.axis_index("c")
        s = lax.axis_index("s")
        wid = c * NSUB + s
        r0 = s * STRIPE
        pltpu.sync_copy(zrows.at[pl.ds(0, CHUNK)], zbuf)
        for off, ln in _PIECES:
            pltpu.sync_copy(zbuf.at[pl.ds(0, ln)], acc.at[pl.ds(r0 + off, ln)])
        plsc.subcore_barrier()

        base = jnp.where(c == 0, s * (C * nch0),
                         NSUB * C * nch0 + s * (C * nch1))
        nquad = jnp.where(c == 0, nch0 // 4, nch1 // 4)

        def issue_idx(p, chunk):
            b = base + chunk * C
            pltpu.async_copy(srcp.at[pl.ds(b, C)], sidx[p], semi[p])
            pltpu.async_copy(dstp.at[pl.ds(b, C)], didx[p], semi[p])

        def wait_idx(p):
            pltpu.make_async_copy(srcp.at[pl.ds(0, C)], sidx[p], semi[p]).wait()
            pltpu.make_async_copy(dstp.at[pl.ds(0, C)], didx[p], semi[p]).wait()

        def wait_scat(x):
            pltpu.make_async_copy(rows[x], acc.at[didx[0]], sems[x]).wait()

        for p in range(4):
            issue_idx(p, p)

        def quad(k, carry):
            for j in range(4):
                cix = 4 * k + j
                x = j % 2
                if j >= 2:
                    wait_scat(x)
                    issue_idx(j - 2, cix + 2)
                else:
                    @pl.when(k > 0)
                    def _():
                        wait_scat(x)
                        issue_idx(j + 2, cix + 2)
                wait_idx(j)
                pltpu.async_copy(y.at[sidx[j]], rows[x], semg).wait()
                pltpu.async_copy(rows[x], acc.at[didx[j]], sems[x], add=True)
            return carry

        lax.fori_loop(0, nquad, quad, 0)
        wait_scat(0)
        wait_scat(1)
        wait_idx(0)
        wait_idx(1)
        plsc.subcore_barrier()

        for off, ln in _PIECES:
            pltpu.sync_copy(acc.at[pl.ds(r0 + off, ln)], zbuf.at[pl.ds(0, ln)])
            pltpu.sync_copy(zbuf.at[pl.ds(0, ln)],
                            out.at[c, pl.ds(r0 + off, ln)])

    return pl.kernel(body, out_type=out_type, mesh=mesh, scratch_types=scratch)


CW = 128


def _make_cnt_kernel():
    mesh = plsc.VectorSubcoreMesh(core_axis_name="c", subcore_axis_name="s")
    out_type = [jax.ShapeDtypeStruct((NSC, NPAD, CW), jnp.float32)]
    scratch = [
        pltpu.VMEM_SHARED((NPAD, CW), jnp.float32),
        [pltpu.VMEM((C,), jnp.int32) for _ in range(4)],
        pltpu.VMEM((C, CW), jnp.float32),
        pltpu.VMEM((CHUNK, CW), jnp.float32),
        [pltpu.SemaphoreType.DMA for _ in range(4)],
        [pltpu.SemaphoreType.DMA for _ in range(2)],
    ]

    def body(dstp, z16, ones_in, cnt_out, acc16, didx, ones_v, b16,
             semi, sems):
        c = lax.axis_index("c")
        s = lax.axis_index("s")
        wid = c * NSUB + s
        r0 = s * STRIPE
        pltpu.sync_copy(z16.at[pl.ds(0, CHUNK)], b16)
        for off, ln in _PIECES:
            pltpu.sync_copy(b16.at[pl.ds(0, ln)], acc16.at[pl.ds(r0 + off, ln)])
        pltpu.sync_copy(ones_in, ones_v)
        plsc.subcore_barrier()

        base = wid * EPT

        def issue_idx(p, chunk):
            pltpu.async_copy(dstp.at[pl.ds(base + chunk * C, C)],
                             didx[p], semi[p])

        def wait_idx(p):
            pltpu.make_async_copy(dstp.at[pl.ds(0, C)], didx[p],
                                  semi[p]).wait()

        def wait_scat(x):
            pltpu.make_async_copy(ones_v, acc16.at[didx[0]], sems[x]).wait()

        for p in range(4):
            issue_idx(p, p)

        def quad(k, carry):
            for j in range(4):
                cix = 4 * k + j
                x = j % 2
                if j >= 2:
                    wait_scat(x)
                    issue_idx(j - 2, cix + 2)
                else:
                    @pl.when(k > 0)
                    def _():
                        wait_scat(x)
                        issue_idx(j + 2, cix + 2)
                wait_idx(j)
                pltpu.async_copy(ones_v, acc16.at[didx[j]], sems[x], add=True)
            return carry

        lax.fori_loop(0, NCH // 4, quad, 0)
        wait_scat(0)
        wait_scat(1)
        wait_idx(0)
        wait_idx(1)
        plsc.subcore_barrier()

        for off, ln in _PIECES:
            pltpu.sync_copy(acc16.at[pl.ds(r0 + off, ln)], b16.at[pl.ds(0, ln)])
            pltpu.sync_copy(b16.at[pl.ds(0, ln)],
                            cnt_out.at[c, pl.ds(r0 + off, ln)])

    return pl.kernel(body, out_type=out_type, mesh=mesh, scratch_types=scratch)


def _softplus(x):
    return jnp.maximum(x, 0.0) + jnp.log(1.0 + jnp.exp(-jnp.abs(x)))


def _mish(x):
    return x * jnp.tanh(_softplus(x))


def _tc_prep_body(x_ref, wl_ref, wr_ref, bl_ref, y_ref, z_ref):
    xx = x_ref[...]
    y_ref[...] = jnp.dot(xx, wl_ref[...], preferred_element_type=jnp.float32)
    z_ref[...] = (jnp.dot(xx, wr_ref[...], preferred_element_type=jnp.float32)
                  + bl_ref[...])


def _norm_act(s_ref, c_ref, z_ref, g_ref, b_ref):
    s = s_ref[...]
    cc = c_ref[...]
    cnt = cc[0, :N, :1] + cc[1, :N, :1]
    h = (s[0, :N] + s[1, :N]) / jnp.maximum(cnt, 1.0) + z_ref[...]
    m = jnp.mean(h, axis=0, keepdims=True)
    v = jnp.mean((h - m) ** 2, axis=0, keepdims=True)
    return (h - m) * lax.rsqrt(v + 1e-5) * g_ref[...] + b_ref[...]


def _tc_mid_body(s_ref, c_ref, z_ref, g_ref, b_ref, wl_ref, wr_ref, bl_ref,
                 y_ref, z2_ref):
    a = _mish(_norm_act(s_ref, c_ref, z_ref, g_ref, b_ref))
    y_ref[...] = jnp.dot(a, wl_ref[...], preferred_element_type=jnp.float32)
    z2_ref[...] = (jnp.dot(a, wr_ref[...], preferred_element_type=jnp.float32)
                   + bl_ref[...])


def _tc_fin_body(s_ref, c_ref, z_ref, g_ref, b_ref,
                 pw1_ref, pb1_ref, pw2_ref, pb2_ref,
                 aw1_ref, ab1_ref, aw2_ref, ab2_ref,
                 pred_ref, emb_ref, aux_ref):
    s = s_ref[...]
    cc = c_ref[...]
    cnt = cc[0, :N, :1] + cc[1, :N, :1]
    h = (s[0, :N, :64] + s[1, :N, :64]) / jnp.maximum(cnt, 1.0) + z_ref[...]
    m = jnp.mean(h, axis=0, keepdims=True)
    v = jnp.mean((h - m) ** 2, axis=0, keepdims=True)
    emb = (h - m) * lax.rsqrt(v + 1e-5) * g_ref[...] + b_ref[...]
    emb_ref[...] = emb
    t = _mish(jnp.dot(emb, pw1_ref[...], preferred_element_type=jnp.float32)
              + pb1_ref[...])
    pred_ref[...] = (jnp.dot(t, pw2_ref[...], preferred_element_type=jnp.float32)
                     + pb2_ref[...])
    u = _mish(jnp.dot(emb, aw1_ref[...], preferred_element_type=jnp.float32)
              + ab1_ref[...])
    aux_ref[...] = (jnp.dot(u, aw2_ref[...], preferred_element_type=jnp.float32)
                    + ab2_ref[...])


def _f32(*shape):
    return jax.ShapeDtypeStruct(shape, jnp.float32)


def kernel(x, edge_index, conv0_Wl, conv0_bl, conv0_Wr, bn0_g, bn0_b,
           conv1_Wl, conv1_bl, conv1_Wr, bn1_g, bn1_b,
           conv2_Wl, conv2_bl, conv2_Wr, bn2_g, bn2_b,
           pred_W1, pred_b1, pred_W2, pred_b2,
           aux_W1, aux_b1, aux_W2, aux_b2):
    src = edge_index[0].astype(jnp.int32)
    dst = edge_index[1].astype(jnp.int32)
    pad = EALLOC - E
    src_p = jnp.concatenate([src, jnp.zeros((pad,), jnp.int32)])
    dst_p = jnp.concatenate([dst, jnp.full((pad,), N, jnp.int32)])
    z128 = jnp.zeros((NPAD, 128), jnp.float32)
    zcw = jnp.zeros((NPAD, CW), jnp.float32)
    onescw = jnp.ones((C, CW), jnp.float32)

    scat = _make_edge_scatter(128, nch0=200, nch1=80)
    cntk = _make_cnt_kernel()

    (c0,) = cntk(dst_p, zcw, onescw)

    y0, z0 = pl.pallas_call(
        _tc_prep_body,
        out_shape=[_f32(N, 128), _f32(N, 128)],
    )(x, conv0_Wl.T, conv0_Wr.T, conv0_bl[None, :])

    (s0,) = scat(y0, src_p, dst_p, z128)

    y1, z1 = pl.pallas_call(
        _tc_mid_body,
        out_shape=[_f32(N, 128), _f32(N, 128)],
    )(s0, c0, z0, bn0_g[None, :], bn0_b[None, :], conv1_Wl.T, conv1_Wr.T,
      conv1_bl[None, :])

    (s1,) = scat(y1, src_p, dst_p, z128)

    wl2T_pad = jnp.pad(conv2_Wl.T, ((0, 0), (0, 64)))
    y2, z2 = pl.pallas_call(
        _tc_mid_body,
        out_shape=[_f32(N, 128), _f32(N, 64)],
    )(s1, c0, z1, bn1_g[None, :], bn1_b[None, :], wl2T_pad, conv2_Wr.T,
      conv2_bl[None, :])

    (s2,) = scat(y2, src_p, dst_p, z128)

    pred, emb, aux = pl.pallas_call(
        _tc_fin_body,
        out_shape=[_f32(N, 1), _f32(N, 64), _f32(N, 1)],
    )(s2, c0, z2, bn2_g[None, :], bn2_b[None, :],
      pred_W1.T, pred_b1[None, :], pred_W2.T, pred_b2[None, :],
      aux_W1.T, aux_b1[None, :], aux_W2.T, aux_b2[None, :])

    return (pred, emb, aux)
